# Initial kernel scaffold; baseline (speedup 1.0000x reference)
import jax
import jax.numpy as jnp
from jax.experimental import pallas as pl


def kernel(X, W, dim):
    raise NotImplementedError("write your pallas kernel here")


# speedup vs baseline: 1.7012x; 1.7012x over previous
import functools

import jax
from jax import lax
from jax.experimental import pallas as pl
from jax.experimental.pallas import tpu as pltpu
from jax.experimental.pallas import tpu_sc as plsc

_NC = 2
_NS = 16


@functools.partial(jax.jit, static_argnums=(0, 1, 2))
def _broadcast_rows(B, T, H, W):
    nw = _NC * _NS
    rows_w = T // nw
    ch = 64 if rows_w % 64 == 0 else rows_w
    n_chunks = rows_w // ch
    mesh = plsc.VectorSubcoreMesh(
        core_axis_name="c", subcore_axis_name="s",
        num_cores=_NC, num_subcores=_NS,
    )

    @functools.partial(
        pl.kernel,
        mesh=mesh,
        out_type=jax.ShapeDtypeStruct((B, T, H), W.dtype),
        scratch_types=[
            pltpu.VMEM((ch, H), W.dtype),
            pltpu.VMEM((ch, H), W.dtype),
            pltpu.SemaphoreType.DMA,
            pltpu.SemaphoreType.DMA,
        ],
    )
    def body(w_hbm, out_hbm, buf0, buf1, lsem, ssem):
        wid = lax.axis_index("s") * _NC + lax.axis_index("c")
        base = wid * rows_w
        bufs = (buf0, buf1)
        loads = [None] * n_chunks
        pending = [[], []]
        loads[0] = pltpu.async_copy(w_hbm.at[pl.ds(base, ch)], bufs[0], lsem)
        for i in range(n_chunks):
            nxt = (i + 1) % 2
            if i + 1 < n_chunks:
                for st in pending[nxt]:
                    st.wait()
                pending[nxt] = []
                loads[i + 1] = pltpu.async_copy(
                    w_hbm.at[pl.ds(base + (i + 1) * ch, ch)], bufs[nxt], lsem)
            loads[i].wait()
            for b in range(B):
                pending[i % 2].append(pltpu.async_copy(
                    bufs[i % 2],
                    out_hbm.at[b].at[pl.ds(base + i * ch, ch)],
                    ssem))
        for lst in pending:
            for st in lst:
                st.wait()

    return body(W)


def kernel(X, W, dim):
    B, T = X.shape
    _, H = W.shape
    return _broadcast_rows(B, T, H, W)
